# Initial kernel scaffold; baseline (speedup 1.0000x reference)
import jax
import jax.numpy as jnp
from jax.experimental import pallas as pl


def kernel(input_coords_cpu, input_resnames, input_atomnames):
    raise NotImplementedError("write your pallas kernel here")


# speedup vs baseline: 12.2151x; 12.2151x over previous
import dataclasses

import jax
import jax.numpy as jnp
from jax import lax
from jax.experimental import pallas as pl
from jax.experimental.pallas import tpu as pltpu
from jax.experimental.pallas import tpu_sc as plsc

_cparams = pltpu.CompilerParams()
if "needs_layout_passes" in pltpu.CompilerParams.__dataclass_fields__:
    _cparams = dataclasses.replace(_cparams, needs_layout_passes=False)
if "use_tc_tiling_on_sc" in pltpu.CompilerParams.__dataclass_fields__:
    _cparams = dataclasses.replace(_cparams, use_tc_tiling_on_sc=False)

N_ATOMS = 1000000
N_TYPES = 11

NC = 2
NS = 16
W = NC * NS
L = 16

C = 31360
HALVES = 2
CH = C // HALVES
HL = CH // L
NP = W * C
PAD = NP - N_ATOMS

HISTW = HALVES * L * N_TYPES
HISTSZ = W * HISTW
NSLOT = W * HALVES * L

BUCKET = CH * 3 + 8 * N_TYPES + 32
SB = 2240
NSB = CH // SB
BFULL = 8192
LADDER = [4096, 2048, 1024, 512, 256, 128, 64, 32, 16, 8]
TIERS = [64, 128, 256, 512]
OUTW = 3 * NP + 256

_mesh = plsc.VectorSubcoreMesh(core_axis_name="c", subcore_axis_name="s")
_i32 = jnp.int32


def _wid():
    return lax.axis_index("s") * NC + lax.axis_index("c")


@jax.jit
def _hist_call(res, atm):
    @pl.kernel(
        out_type=(
            jax.ShapeDtypeStruct((NP,), _i32),
            jax.ShapeDtypeStruct((HISTSZ,), _i32),
        ),
        mesh=_mesh,
        compiler_params=_cparams,
        scratch_types=[
            pltpu.VMEM((C,), _i32),
            pltpu.VMEM((C,), _i32),
            pltpu.VMEM((C,), _i32),
            pltpu.VMEM((HISTW,), _i32),
        ],
    )
    def k(res_hbm, atm_hbm, types_hbm, hist_hbm, res_v, atm_v, types_v, cnt_v):
        w = _wid()
        base = w * C
        pltpu.sync_copy(res_hbm.at[pl.ds(base, C)], res_v)
        pltpu.sync_copy(atm_hbm.at[pl.ds(base, C)], atm_v)

        zeros = jnp.zeros((L,), _i32)
        for kk in range(HISTW // L):
            cnt_v[pl.ds(kk * L, L)] = zeros

        @pl.loop(0, C, step=L)
        def _(i):
            r = res_v[pl.ds(i, L)]
            a = atm_v[pl.ds(i, L)]
            types_v[pl.ds(i, L)] = (r * 7 + a) % N_TYPES

        lane = lax.iota(_i32, L)
        ones = jnp.ones((L,), _i32)

        for h in range(HALVES):
            @pl.loop(0, HL)
            def _(i, h=h):
                t = plsc.load_gather(types_v, [h * CH + lane * HL + i])
                plsc.addupdate_scatter(
                    cnt_v, [(h * L + lane) * N_TYPES + t], ones)

        pltpu.sync_copy(types_v, types_hbm.at[pl.ds(base, C)])
        pltpu.sync_copy(cnt_v, hist_hbm.at[pl.ds(w * HISTW, HISTW)])

    return k(res, atm)


@jax.jit
def _scatter_call(coords, types, hist):
    @pl.kernel(
        out_type=(
            jax.ShapeDtypeStruct((OUTW,), jnp.float32),
            jax.ShapeDtypeStruct((L,), _i32),
            jax.ShapeDtypeStruct((L,), _i32),
        ),
        mesh=_mesh,
        compiler_params=_cparams,
        scratch_types=[
            pltpu.VMEM((CH,), _i32),
            pltpu.VMEM((BUCKET,), jnp.float32),
            pltpu.VMEM((HISTSZ,), _i32),
            pltpu.VMEM((N_TYPES * NSLOT,), _i32),
            pltpu.VMEM((HALVES * L * N_TYPES,), _i32),
            pltpu.VMEM((L,), _i32),
            pltpu.VMEM((L,), _i32),
            pltpu.VMEM((512,), jnp.float32),
            pltpu.VMEM((64,), _i32),
            pltpu.VMEM((128,), _i32),
            pltpu.VMEM((256,), _i32),
            pltpu.VMEM((512,), _i32),
            pltpu.VMEM((SB * 3,), jnp.float32),
            pltpu.VMEM((SB * 3,), jnp.float32),
            pltpu.SemaphoreType.DMA,
            pltpu.SemaphoreType.DMA,
            pltpu.SemaphoreType.DMA,
        ],
    )
    def k(coords_hbm, types_hbm, hist_hbm, out_hbm, cnt_hbm, off_hbm,
          types_h, bucket_v, hist_v, basecol_v, state_v, co_v, of_v,
          st_v, wi64_v, wi128_v, wi256_v, wi512_v, crd0_v, crd1_v,
          semA, semB, semW):
        w = _wid()
        lane = lax.iota(_i32, L)
        wi_tiers = list(zip(TIERS, [wi64_v, wi128_v, wi256_v, wi512_v]))

        pltpu.sync_copy(hist_hbm, hist_v)

        trash = 3 * NP + w * 8 + lax.bitwise_and(lane, 7)
        for tsz, tref in wi_tiers:
            for b in range(tsz // L):
                tref[pl.ds(b * L, L)] = trash

        G = jnp.int32(0)
        counts_vec = jnp.zeros((L,), _i32)
        offs_vec = jnp.zeros((L,), _i32)
        for t in range(N_TYPES):
            offs_vec = offs_vec + jnp.where(lane == t, G, 0)

            def g_body(g, run, t=t):
                v = plsc.load_gather(hist_v, [(g * L + lane) * N_TYPES + t])
                c = jnp.cumsum(v)
                plsc.store_scatter(
                    basecol_v, [t * NSLOT + g * L + lane], run + c - v)
                return run + jnp.sum(v)

            tot = lax.fori_loop(0, NSLOT // L, g_body, G)
            counts_vec = counts_vec + jnp.where(lane == t, tot - G, 0)
            G = tot

        @pl.when(w == 0)
        def _():
            co_v[...] = counts_vec - jnp.where(lane == (N_TYPES - 1), PAD, 0)
            of_v[...] = offs_vec
            pltpu.sync_copy(co_v, cnt_hbm)
            pltpu.sync_copy(of_v, off_hbm)

        htc = jnp.int32(0)
        for h in range(HALVES):
            slot0 = (w * HALVES + h) * L
            hbase = w * C + h * CH

            pltpu.sync_copy(types_hbm.at[pl.ds(hbase, CH)], types_h)

            pe = jnp.int32(0)
            run_info = []
            for t in range(N_TYPES):
                v = plsc.load_gather(hist_v, [(slot0 + lane) * N_TYPES + t])
                c = jnp.cumsum(v)
                cnt = jnp.sum(v)
                gb = plsc.load_gather(
                    basecol_v,
                    [jnp.full((L,), t * NSLOT, _i32) + slot0 + lane * 0])
                gbase = jnp.max(gb)
                ws = gbase * 3
                we = ws + cnt * 3
                lw = pe + lax.bitwise_and(ws - pe, 7)
                pe = lw + cnt * 3
                seeds = lw + 3 * (c - v)
                plsc.store_scatter(
                    state_v, [(h * L + lane) * N_TYPES + t], seeds)
                run_info.append((ws, we, lw))

            threes = jnp.full((L,), 3, _i32)

            @pl.loop(0, HL)
            def _(i, h=h):
                f = lane * HL + i
                t = plsc.load_gather(types_h, [f])
                sidx = (h * L + lane) * N_TYPES + t
                p = plsc.load_gather(state_v, [sidx])
                plsc.addupdate_scatter(state_v, [sidx], threes)
                plsc.store_scatter(types_h, [f], p)

            def issue(s, buf, sem):
                return pltpu.async_copy(
                    coords_hbm.at[pl.ds((hbase + s * SB) * 3, SB * 3)],
                    buf, sem)

            issue(0, crd0_v, semA)
            for s in range(NSB):
                buf, sem = (crd0_v, semA) if s % 2 == 0 else (crd1_v, semB)
                if s + 1 < NSB:
                    nbuf, nsem = (crd1_v, semB) if s % 2 == 0 else (crd0_v, semA)
                    issue(s + 1, nbuf, nsem)
                pltpu.make_async_copy(
                    coords_hbm.at[pl.ds((hbase + s * SB) * 3, SB * 3)],
                    buf, sem).wait()

                @pl.loop(0, SB // L)
                def _(kk, s=s, buf=buf):
                    q = types_h[pl.ds(s * SB + kk * L, L)]
                    for cc in range(3):
                        val = plsc.load_gather(buf, [(kk * L + lane) * 3 + cc])
                        plsc.store_scatter(bucket_v, [q + cc], val)

            mid_total = jnp.int32(0)
            for t in range(N_TYPES):
                ws, we, lw = run_info[t]
                s8 = lax.bitwise_and(ws + 7, -8)
                e8 = lax.bitwise_and(we, -8)
                me = jnp.maximum(e8, s8)
                mlen = me - s8
                b8 = lw + (s8 - ws)
                nb = lax.shift_right_logical(mlen, 13)
                rem = lax.bitwise_and(mlen, BFULL - 1)

                def mid_body(kk, _, b8=b8, s8=s8):
                    pltpu.async_copy(
                        bucket_v.at[pl.ds(
                            pl.multiple_of(b8 + kk * BFULL, 8), BFULL)],
                        out_hbm.at[pl.ds(
                            pl.multiple_of(s8 + kk * BFULL, 8), BFULL)],
                        semW)
                    return 0

                lax.fori_loop(0, nb, mid_body, 0)
                for sz in LADDER:
                    @pl.when(lax.bitwise_and(rem, sz) != 0)
                    def _(sz=sz, rem=rem, nb=nb, b8=b8, s8=s8):
                        off = nb * BFULL + lax.bitwise_and(rem, -(2 * sz))
                        pltpu.async_copy(
                            bucket_v.at[pl.ds(
                                pl.multiple_of(b8 + off, 8), sz)],
                            out_hbm.at[pl.ds(
                                pl.multiple_of(s8 + off, 8), sz)],
                            semW)
                mid_total = mid_total + mlen

                hlen = jnp.minimum(s8, we) - ws
                tstart = jnp.maximum(s8, e8)
                tlen = jnp.maximum(we - tstart, 0)
                for sstart, slen in ((ws, hlen), (tstart, tlen)):
                    boff = lw + (sstart - ws)
                    wordsv = plsc.load_gather(bucket_v, [boff + lane])
                    valid = lane < slen
                    plsc.store_scatter(
                        st_v, [htc + lane], wordsv, mask=valid)
                    idxv = sstart + lane
                    for tsz, tref in wi_tiers:
                        plsc.store_scatter(
                            tref, [htc + lane], idxv,
                            mask=jnp.logical_and(valid, (htc + lane) < tsz))
                    htc = htc + slen

            def drain_body(kk, _):
                pltpu.make_async_copy(
                    bucket_v.at[pl.ds(0, BFULL)],
                    out_hbm.at[pl.ds(0, BFULL)], semW).wait()
                return 0

            lax.fori_loop(
                0, lax.shift_right_logical(mid_total, 13), drain_body, 0)
            for sz in LADDER:
                @pl.when(lax.bitwise_and(mid_total, sz) != 0)
                def _(sz=sz):
                    pltpu.make_async_copy(
                        bucket_v.at[pl.ds(0, sz)],
                        out_hbm.at[pl.ds(0, sz)], semW).wait()

        done = jnp.bool_(False)
        for tsz, tref in wi_tiers:
            fits = jnp.logical_and(htc <= tsz, jnp.logical_not(done))

            @pl.when(fits)
            def _(tsz=tsz, tref=tref):
                pltpu.sync_copy(st_v.at[pl.ds(0, tsz)], out_hbm.at[tref])

            done = jnp.logical_or(done, htc <= tsz)

    return k(coords, types, hist)


def kernel(input_coords_cpu, input_resnames, input_atomnames):
    res = input_resnames.astype(_i32)
    atm = input_atomnames.astype(_i32)
    res_p = jnp.concatenate([res, jnp.zeros((PAD,), _i32)])
    atm_p = jnp.concatenate([atm, jnp.full((PAD,), 10, _i32)])
    coords_p = jnp.concatenate([
        input_coords_cpu.astype(jnp.float32),
        jnp.zeros((3 * PAD,), jnp.float32),
    ])

    types, hist = _hist_call(res_p, atm_p)
    out, counts, offsets = _scatter_call(coords_p, types, hist)
    return (
        out[:3 * N_ATOMS],
        counts[:N_TYPES],
        offsets[:N_TYPES],
    )
